# Initial kernel scaffold; baseline (speedup 1.0000x reference)
import jax
import jax.numpy as jnp
from jax.experimental import pallas as pl


def kernel(x, edge_index, batch, W1, b1, W2, b2, W3, b3, Wq, bq, Wk, bk, Wv, bv, Wo, bo, ln1_g, ln1_b, Wf1, bf1, Wf2, bf2, ln2_g, ln2_b):
    raise NotImplementedError("write your pallas kernel here")


# speedup vs baseline: 17.1436x; 17.1436x over previous
import functools
import jax
import jax.numpy as jnp
from jax import lax
from jax.experimental import pallas as pl
from jax.experimental.pallas import tpu as pltpu
from jax.experimental.pallas import tpu_sc as plsc

_N = 50000
_E = 800000
_EB = 128
_NSUB = 16
_NCORE = 2
_EPAD = 802816
_EC = _EPAD // _NCORE
_TB_SPLIT = _EC // (_NSUB * _EB)
_TB_ALL = _EPAD // (_NSUB * _EB)
_NP = 50176
_RPT = _NP // _NSUB
_RB = 1000
_G = _N // _RB

_mesh = plsc.VectorSubcoreMesh(core_axis_name="c", subcore_axis_name="s")
_sc_params = pltpu.CompilerParams(use_tc_tiling_on_sc=False)


def _wid(): return lax.axis_index("s")


def _cid(): return lax.axis_index("c")


@functools.partial(
    pl.kernel, mesh=_mesh, compiler_params=_sc_params,
    out_type=jax.ShapeDtypeStruct((_NCORE, _NP, 16), jnp.float32),
    scratch_types=[
        pltpu.VMEM((_EB,), jnp.int32),
        pltpu.VMEM((_EB, 16), jnp.float32),
        pltpu.VMEM_SHARED((_NP, 16), jnp.float32),
    ],
)
def _sc_deg(dst_hbm, ones_hbm, z_hbm, out_hbm, idxd, ones_v, acc):
    sid, cid = _wid(), _cid()
    r0 = sid * _RPT
    pltpu.sync_copy(ones_hbm, ones_v)
    pltpu.sync_copy(z_hbm.at[pl.ds(r0, _RPT)], acc.at[pl.ds(r0, _RPT)])
    plsc.subcore_barrier()
    e0 = cid * _EC + sid * (_TB_SPLIT * _EB)

    def body(b, carry):
        base = e0 + b * _EB
        pltpu.sync_copy(dst_hbm.at[pl.ds(base, _EB)], idxd)
        pltpu.sync_copy(ones_v, acc.at[idxd], add=True)
        return carry

    lax.fori_loop(0, _TB_SPLIT, body, 0)
    plsc.subcore_barrier()
    pltpu.sync_copy(acc.at[pl.ds(r0, _RPT)], out_hbm.at[cid, pl.ds(r0, _RPT)])


@functools.partial(
    pl.kernel, mesh=_mesh, compiler_params=_sc_params,
    out_type=jax.ShapeDtypeStruct((_NCORE, _NP, 16), jnp.float32),
    scratch_types=[
        pltpu.VMEM((_EB,), jnp.int32),
        pltpu.VMEM((_EB,), jnp.int32),
        pltpu.VMEM((_EB, 16), jnp.float32),
        pltpu.VMEM_SHARED((_NP, 16), jnp.float32),
        pltpu.SemaphoreType.DMA,
    ],
)
def _sc_agg16(src_hbm, dst_hbm, tab_hbm, z_hbm, out_hbm,
              idxs, idxd, rows, acc, sem):
    sid, cid = _wid(), _cid()
    r0 = sid * _RPT
    pltpu.sync_copy(z_hbm.at[pl.ds(r0, _RPT)], acc.at[pl.ds(r0, _RPT)])
    plsc.subcore_barrier()
    e0 = cid * _EC + sid * (_TB_SPLIT * _EB)

    def body(b, carry):
        base = e0 + b * _EB
        pltpu.sync_copy(src_hbm.at[pl.ds(base, _EB)], idxs)
        pltpu.sync_copy(dst_hbm.at[pl.ds(base, _EB)], idxd)
        pltpu.async_copy(tab_hbm.at[idxs], rows, sem).wait()
        pltpu.sync_copy(rows, acc.at[idxd], add=True)
        return carry

    lax.fori_loop(0, _TB_SPLIT, body, 0)
    plsc.subcore_barrier()
    pltpu.sync_copy(acc.at[pl.ds(r0, _RPT)], out_hbm.at[cid, pl.ds(r0, _RPT)])


@functools.partial(
    pl.kernel, mesh=_mesh, compiler_params=_sc_params,
    out_type=jax.ShapeDtypeStruct((_NCORE, _NP, 32), jnp.float32),
    scratch_types=[
        pltpu.VMEM((_EB,), jnp.int32),
        pltpu.VMEM((_EB,), jnp.int32),
        pltpu.VMEM((_EB, 32), jnp.float32),
        pltpu.VMEM_SHARED((_NP, 32), jnp.float32),
        pltpu.SemaphoreType.DMA,
    ],
)
def _sc_agg64(src_hbm, dst_hbm, tlo_hbm, thi_hbm, z_hbm, out_hbm,
              idxs, idxd, rows, acc, sem):
    sid, cid = _wid(), _cid()
    r0 = sid * _RPT
    pltpu.sync_copy(z_hbm.at[pl.ds(r0, _RPT)], acc.at[pl.ds(r0, _RPT)])
    plsc.subcore_barrier()
    e0 = sid * (_TB_ALL * _EB)

    def body(b, carry):
        base = e0 + b * _EB
        pltpu.sync_copy(src_hbm.at[pl.ds(base, _EB)], idxs)
        pltpu.sync_copy(dst_hbm.at[pl.ds(base, _EB)], idxd)

        @pl.when(cid == 0)
        def _():
            pltpu.async_copy(tlo_hbm.at[idxs], rows, sem).wait()

        @pl.when(cid == 1)
        def _():
            pltpu.async_copy(thi_hbm.at[idxs], rows, sem).wait()

        pltpu.sync_copy(rows, acc.at[idxd], add=True)
        return carry

    lax.fori_loop(0, _TB_ALL, body, 0)
    plsc.subcore_barrier()
    pltpu.sync_copy(acc.at[pl.ds(r0, _RPT)], out_hbm.at[cid, pl.ds(r0, _RPT)])


@functools.partial(
    pl.kernel, mesh=_mesh, compiler_params=_sc_params,
    out_type=[jax.ShapeDtypeStruct((_NCORE, _NP, 16), jnp.float32),
              jax.ShapeDtypeStruct((_NCORE, _NP, 16), jnp.float32)],
    scratch_types=[
        pltpu.VMEM((_EB,), jnp.int32),
        pltpu.VMEM((_EB,), jnp.int32),
        pltpu.VMEM((_EB, 16), jnp.float32),
        pltpu.VMEM((_EB, 16), jnp.float32),
        pltpu.VMEM((_EB, 16), jnp.float32),
        pltpu.VMEM((_EB, 16), jnp.float32),
        pltpu.VMEM((_EB, 16), jnp.float32),
        pltpu.VMEM((16,), jnp.float32),
        pltpu.VMEM((16,), jnp.float32),
        pltpu.VMEM_SHARED((_NP, 16), jnp.float32),
        pltpu.VMEM_SHARED((_NP, 16), jnp.float32),
        pltpu.SemaphoreType.DMA,
        pltpu.SemaphoreType.DMA,
        pltpu.SemaphoreType.DMA,
    ],
)
def _sc_attn(src_hbm, dst_hbm, q_hbm, k_hbm, v_hbm, mq_hbm, mk_hbm, z_hbm,
             oe_hbm, oev_hbm,
             idxs, idxd, qr, kr, vr, eb, evb, mq, mk, acc_e, acc_ev,
             sem_q, sem_k, sem_v):
    sid, cid = _wid(), _cid()
    r0 = sid * _RPT
    pltpu.sync_copy(mq_hbm, mq)
    pltpu.sync_copy(mk_hbm, mk)
    pltpu.sync_copy(z_hbm.at[pl.ds(r0, _RPT)], acc_e.at[pl.ds(r0, _RPT)])
    pltpu.sync_copy(z_hbm.at[pl.ds(r0, _RPT)], acc_ev.at[pl.ds(r0, _RPT)])
    plsc.subcore_barrier()
    bvec = mq[...] * mk[...]
    e0 = cid * _EC + sid * (_TB_SPLIT * _EB)

    def body(b, carry):
        base = e0 + b * _EB
        pltpu.sync_copy(src_hbm.at[pl.ds(base, _EB)], idxs)
        pltpu.sync_copy(dst_hbm.at[pl.ds(base, _EB)], idxd)
        cq = pltpu.async_copy(q_hbm.at[idxd], qr, sem_q)
        ck = pltpu.async_copy(k_hbm.at[idxs], kr, sem_k)
        cv = pltpu.async_copy(v_hbm.at[idxs], vr, sem_v)
        cq.wait()
        ck.wait()
        cv.wait()

        def inner(j, c):
            e = jnp.exp(qr[j] * kr[j] - bvec)
            eb[j] = e
            evb[j] = e * vr[j]
            return c

        lax.fori_loop(0, _EB, inner, 0)
        pltpu.sync_copy(eb, acc_e.at[idxd], add=True)
        pltpu.sync_copy(evb, acc_ev.at[idxd], add=True)
        return carry

    lax.fori_loop(0, _TB_SPLIT, body, 0)
    plsc.subcore_barrier()
    pltpu.sync_copy(acc_e.at[pl.ds(r0, _RPT)], oe_hbm.at[cid, pl.ds(r0, _RPT)])
    pltpu.sync_copy(acc_ev.at[pl.ds(r0, _RPT)],
                    oev_hbm.at[cid, pl.ds(r0, _RPT)])


def _ln(x, g, b):
    m = jnp.mean(x, axis=-1, keepdims=True)
    v = jnp.mean((x - m) * (x - m), axis=-1, keepdims=True)
    return (x - m) * lax.rsqrt(v + 1e-5) * g + b


def _t0_body(deg_ref, x_ref, dinv_ref, xs_ref):
    d = deg_ref[0, :, :1] + deg_ref[1, :, :1] + 1.0
    di = lax.rsqrt(d)
    dinv_ref[...] = di
    xs_ref[...] = jnp.concatenate(
        [di * x_ref[...], jnp.zeros((_RB, 8), jnp.float32)], axis=1)


def _t1_body(p1_ref, xs_ref, dinv_ref, w1_ref, b1_ref, lo_ref, hi_ref):
    di = dinv_ref[...]
    agg16 = di * (p1_ref[0] + p1_ref[1] + xs_ref[...])
    h1 = jax.nn.relu(agg16[:, :8] @ w1_ref[...] + b1_ref[...])
    h1s = di * h1
    lo_ref[...] = h1s[:, :32]
    hi_ref[...] = h1s[:, 32:]


def _t2_body(p2_ref, lo_ref, hi_ref, dinv_ref, w2_ref, b2_ref, w3_ref,
             b3_ref, wq_ref, bq_ref, wk_ref, bk_ref, wv_ref, bv_ref,
             h_ref, q_ref, k_ref, v_ref, mq_ref, mk_ref):
    di = dinv_ref[...]
    agg64 = di * (jnp.concatenate([p2_ref[0], p2_ref[1]], axis=1)
                  + jnp.concatenate([lo_ref[...], hi_ref[...]], axis=1))
    h2 = jax.nn.relu(agg64 @ w2_ref[...] + b2_ref[...])
    h = h2 @ w3_ref[...] + b3_ref[...]
    q = h @ wq_ref[...] + bq_ref[...]
    k = h @ wk_ref[...] + bk_ref[...]
    v = h @ wv_ref[...] + bv_ref[...]
    z = jnp.zeros((_RB, 11), jnp.float32)
    h_ref[...] = h
    q_ref[...] = jnp.concatenate([q, z], axis=1)
    k_ref[...] = jnp.concatenate([k, z], axis=1)
    v_ref[...] = jnp.concatenate([v, z], axis=1)
    z1 = jnp.zeros((1, 11), jnp.float32)
    qm = jnp.concatenate([jnp.max(jnp.abs(q), axis=0, keepdims=True), z1], 1)
    km = jnp.concatenate([jnp.max(jnp.abs(k), axis=0, keepdims=True), z1], 1)
    i = pl.program_id(0)

    @pl.when(i == 0)
    def _():
        mq_ref[...] = qm
        mk_ref[...] = km

    @pl.when(i > 0)
    def _():
        mq_ref[...] = jnp.maximum(mq_ref[...], qm)
        mk_ref[...] = jnp.maximum(mk_ref[...], km)


def _t3_body(oe_ref, oev_ref, h_ref, q_ref, k_ref, v_ref, mq_ref, mk_ref,
             wo_ref, bo_ref, g1_ref, be1_ref, wf1_ref, bf1_ref, wf2_ref,
             bf2_ref, g2_ref, be2_ref, out_ref):
    bvec = mq_ref[...] * mk_ref[...]
    es = jnp.exp(q_ref[...] * k_ref[...] - bvec)
    s5 = oe_ref[0, :, :5] + oe_ref[1, :, :5] + es[:, :5]
    ev5 = (oev_ref[0, :, :5] + oev_ref[1, :, :5]
           + es[:, :5] * v_ref[...][:, :5])
    ctx = ev5 / s5
    o = ctx @ wo_ref[...] + bo_ref[...]
    h1n = _ln(h_ref[...] + o, g1_ref[...], be1_ref[...])
    ff = jax.nn.relu(h1n @ wf1_ref[...] + bf1_ref[...]) @ wf2_ref[...] \
        + bf2_ref[...]
    out_ref[...] = _ln(h1n + ff, g2_ref[...], be2_ref[...])


def _row_bs(width):
    return pl.BlockSpec((_RB, width), lambda i: (i, 0))


def _pair_bs(width):
    return pl.BlockSpec((2, _RB, width), lambda i: (0, i, 0))


def _full_bs(shape):
    nd = len(shape)
    return pl.BlockSpec(shape, lambda i: (0,) * nd)


def _tc_call(body, in_specs, out_specs, out_shapes):
    return pl.pallas_call(
        body, grid=(_G,), in_specs=in_specs, out_specs=out_specs,
        out_shape=out_shapes)


def kernel(x, edge_index, batch, W1, b1, W2, b2, W3, b3, Wq, bq, Wk, bk,
           Wv, bv, Wo, bo, ln1_g, ln1_b, Wf1, bf1, Wf2, bf2, ln2_g, ln2_b):
    f32 = jnp.float32
    src = jnp.concatenate(
        [edge_index[0], jnp.zeros((_EPAD - _E,), jnp.int32)])
    dst = jnp.concatenate(
        [edge_index[1], jnp.full((_EPAD - _E,), _N, jnp.int32)])
    ones128 = jnp.ones((_EB, 16), f32)
    z16 = jnp.zeros((_NP, 16), f32)
    z32 = jnp.zeros((_NP, 32), f32)
    b1r = b1.reshape(1, -1)
    b2r = b2.reshape(1, -1)
    b3r = b3.reshape(1, -1)
    bqr, bkr, bvr, bor = (t.reshape(1, -1) for t in (bq, bk, bv, bo))
    bf1r, bf2r = bf1.reshape(1, -1), bf2.reshape(1, -1)
    g1r, be1r = ln1_g.reshape(1, -1), ln1_b.reshape(1, -1)
    g2r, be2r = ln2_g.reshape(1, -1), ln2_b.reshape(1, -1)

    deg2 = _sc_deg(dst, ones128, z16)

    dinv, xs16 = _tc_call(
        _t0_body,
        [_pair_bs(16), _row_bs(8)],
        [_row_bs(1), _row_bs(16)],
        [jax.ShapeDtypeStruct((_N, 1), f32),
         jax.ShapeDtypeStruct((_N, 16), f32)],
    )(deg2[:, :_N], x)

    p1 = _sc_agg16(src, dst, xs16, z16)

    h1lo, h1hi = _tc_call(
        _t1_body,
        [_pair_bs(16), _row_bs(16), _row_bs(1), _full_bs((8, 64)),
         _full_bs((1, 64))],
        [_row_bs(32), _row_bs(32)],
        [jax.ShapeDtypeStruct((_N, 32), f32),
         jax.ShapeDtypeStruct((_N, 32), f32)],
    )(p1[:, :_N], xs16, dinv, W1, b1r)

    p2 = _sc_agg64(src, dst, h1lo, h1hi, z32)

    h5, q16, k16, v16, mq, mk = _tc_call(
        _t2_body,
        [_pair_bs(32), _row_bs(32), _row_bs(32), _row_bs(1),
         _full_bs((64, 150)), _full_bs((1, 150)), _full_bs((150, 5)),
         _full_bs((1, 5)), _full_bs((5, 5)), _full_bs((1, 5)),
         _full_bs((5, 5)), _full_bs((1, 5)), _full_bs((5, 5)),
         _full_bs((1, 5))],
        [_row_bs(5), _row_bs(16), _row_bs(16), _row_bs(16),
         _full_bs((1, 16)), _full_bs((1, 16))],
        [jax.ShapeDtypeStruct((_N, 5), f32),
         jax.ShapeDtypeStruct((_N, 16), f32),
         jax.ShapeDtypeStruct((_N, 16), f32),
         jax.ShapeDtypeStruct((_N, 16), f32),
         jax.ShapeDtypeStruct((1, 16), f32),
         jax.ShapeDtypeStruct((1, 16), f32)],
    )(p2[:, :_N], h1lo, h1hi, dinv, W2, b2r, W3, b3r, Wq, bqr, Wk, bkr,
      Wv, bvr)

    oe, oev = _sc_attn(src, dst, q16, k16, v16, mq.reshape(16),
                       mk.reshape(16), z16)

    out = _tc_call(
        _t3_body,
        [_pair_bs(16), _pair_bs(16), _row_bs(5), _row_bs(16), _row_bs(16),
         _row_bs(16), _full_bs((1, 16)), _full_bs((1, 16)),
         _full_bs((5, 5)), _full_bs((1, 5)), _full_bs((1, 5)),
         _full_bs((1, 5)), _full_bs((5, 64)), _full_bs((1, 64)),
         _full_bs((64, 5)), _full_bs((1, 5)), _full_bs((1, 5)),
         _full_bs((1, 5))],
        [_row_bs(5)],
        [jax.ShapeDtypeStruct((_N, 5), f32)],
    )(oe[:, :_N], oev[:, :_N], h5, q16, k16, v16, mq, mk, Wo, bor, g1r,
      be1r, Wf1, bf1r, Wf2, bf2r, g2r, be2r)
    return out[0]


# speedup vs baseline: 22.9224x; 1.3371x over previous
import functools
import jax
import jax.numpy as jnp
from jax import lax
from jax.experimental import pallas as pl
from jax.experimental.pallas import tpu as pltpu
from jax.experimental.pallas import tpu_sc as plsc

_N = 50000
_E = 800000
_EB = 128
_NSUB = 16
_NCORE = 2
_EPAD = 802816
_EC = _EPAD // _NCORE
_TB_SPLIT = _EC // (_NSUB * _EB)
_TB_ALL = _EPAD // (_NSUB * _EB)
_NP = 50176
_RPT = _NP // _NSUB
_RB = 1000
_G = _N // _RB

_mesh = plsc.VectorSubcoreMesh(core_axis_name="c", subcore_axis_name="s")
_sc_params = pltpu.CompilerParams(use_tc_tiling_on_sc=False)


def _wid(): return lax.axis_index("s")


def _cid(): return lax.axis_index("c")


@functools.partial(
    pl.kernel, mesh=_mesh, compiler_params=_sc_params,
    out_type=jax.ShapeDtypeStruct((_NCORE, _NP, 16), jnp.float32),
    scratch_types=[
        pltpu.VMEM((_EB,), jnp.int32),
        pltpu.VMEM((_EB, 16), jnp.float32),
        pltpu.VMEM_SHARED((_NP, 16), jnp.float32),
    ],
)
def _sc_deg(dst_hbm, ones_hbm, z_hbm, out_hbm, idxd, ones_v, acc):
    sid, cid = _wid(), _cid()
    r0 = sid * _RPT
    pltpu.sync_copy(ones_hbm, ones_v)
    pltpu.sync_copy(z_hbm.at[pl.ds(r0, _RPT)], acc.at[pl.ds(r0, _RPT)])
    plsc.subcore_barrier()
    e0 = cid * _EC + sid * (_TB_SPLIT * _EB)

    def body(b, carry):
        base = e0 + b * _EB
        pltpu.sync_copy(dst_hbm.at[pl.ds(base, _EB)], idxd)
        pltpu.sync_copy(ones_v, acc.at[idxd], add=True)
        return carry

    lax.fori_loop(0, _TB_SPLIT, body, 0)
    plsc.subcore_barrier()
    pltpu.sync_copy(acc.at[pl.ds(r0, _RPT)], out_hbm.at[cid, pl.ds(r0, _RPT)])


@functools.partial(
    pl.kernel, mesh=_mesh, compiler_params=_sc_params,
    out_type=jax.ShapeDtypeStruct((_NCORE, _NP, 16), jnp.float32),
    scratch_types=[
        pltpu.VMEM((_EB,), jnp.int32),
        pltpu.VMEM((_EB,), jnp.int32),
        pltpu.VMEM((_EB,), jnp.int32),
        pltpu.VMEM((_EB,), jnp.int32),
        pltpu.VMEM((_EB, 16), jnp.float32),
        pltpu.VMEM((_EB, 16), jnp.float32),
        pltpu.VMEM_SHARED((_NP, 16), jnp.float32),
        pltpu.SemaphoreType.DMA,
        pltpu.SemaphoreType.DMA,
    ],
)
def _sc_agg16(src_hbm, dst_hbm, tab_hbm, z_hbm, out_hbm,
              idxs_a, idxd_a, idxs_b, idxd_b, rows_a, rows_b, acc,
              sem_a, sem_b):
    sid, cid = _wid(), _cid()
    r0 = sid * _RPT
    pltpu.sync_copy(z_hbm.at[pl.ds(r0, _RPT)], acc.at[pl.ds(r0, _RPT)])
    plsc.subcore_barrier()
    e0 = cid * _EC + sid * (_TB_SPLIT * _EB)
    np2 = _TB_SPLIT // 2

    pltpu.sync_copy(src_hbm.at[pl.ds(e0, _EB)], idxs_a)
    pltpu.sync_copy(dst_hbm.at[pl.ds(e0, _EB)], idxd_a)
    pltpu.async_copy(tab_hbm.at[idxs_a], rows_a, sem_a)

    def body(p, carry):
        b1 = e0 + (2 * p + 1) * _EB
        pltpu.sync_copy(src_hbm.at[pl.ds(b1, _EB)], idxs_b)
        pltpu.sync_copy(dst_hbm.at[pl.ds(b1, _EB)], idxd_b)
        pltpu.async_copy(tab_hbm.at[idxs_b], rows_b, sem_b)
        pltpu.make_async_copy(tab_hbm.at[idxs_a], rows_a, sem_a).wait()
        pltpu.sync_copy(rows_a, acc.at[idxd_a], add=True)

        @pl.when(p < np2 - 1)
        def _():
            b2 = e0 + (2 * p + 2) * _EB
            pltpu.sync_copy(src_hbm.at[pl.ds(b2, _EB)], idxs_a)
            pltpu.sync_copy(dst_hbm.at[pl.ds(b2, _EB)], idxd_a)
            pltpu.async_copy(tab_hbm.at[idxs_a], rows_a, sem_a)

        pltpu.make_async_copy(tab_hbm.at[idxs_b], rows_b, sem_b).wait()
        pltpu.sync_copy(rows_b, acc.at[idxd_b], add=True)
        return carry

    lax.fori_loop(0, np2, body, 0)
    plsc.subcore_barrier()
    pltpu.sync_copy(acc.at[pl.ds(r0, _RPT)], out_hbm.at[cid, pl.ds(r0, _RPT)])


@functools.partial(
    pl.kernel, mesh=_mesh, compiler_params=_sc_params,
    out_type=jax.ShapeDtypeStruct((_NCORE, _NP, 32), jnp.float32),
    scratch_types=[
        pltpu.VMEM((_EB,), jnp.int32),
        pltpu.VMEM((_EB,), jnp.int32),
        pltpu.VMEM((_EB,), jnp.int32),
        pltpu.VMEM((_EB,), jnp.int32),
        pltpu.VMEM((_EB, 32), jnp.float32),
        pltpu.VMEM((_EB, 32), jnp.float32),
        pltpu.VMEM_SHARED((_NP, 32), jnp.float32),
        pltpu.SemaphoreType.DMA,
        pltpu.SemaphoreType.DMA,
    ],
)
def _sc_agg64(src_hbm, dst_hbm, tlo_hbm, thi_hbm, z_hbm, out_hbm,
              idxs_a, idxd_a, idxs_b, idxd_b, rows_a, rows_b, acc,
              sem_a, sem_b):
    sid, cid = _wid(), _cid()
    r0 = sid * _RPT
    pltpu.sync_copy(z_hbm.at[pl.ds(r0, _RPT)], acc.at[pl.ds(r0, _RPT)])
    plsc.subcore_barrier()
    e0 = sid * (_TB_ALL * _EB)
    np2 = _TB_ALL // 2

    def fire(idxs, rows, sem):
        @pl.when(cid == 0)
        def _():
            pltpu.async_copy(tlo_hbm.at[idxs], rows, sem)

        @pl.when(cid == 1)
        def _():
            pltpu.async_copy(thi_hbm.at[idxs], rows, sem)

    pltpu.sync_copy(src_hbm.at[pl.ds(e0, _EB)], idxs_a)
    pltpu.sync_copy(dst_hbm.at[pl.ds(e0, _EB)], idxd_a)
    fire(idxs_a, rows_a, sem_a)

    def body(p, carry):
        b1 = e0 + (2 * p + 1) * _EB
        pltpu.sync_copy(src_hbm.at[pl.ds(b1, _EB)], idxs_b)
        pltpu.sync_copy(dst_hbm.at[pl.ds(b1, _EB)], idxd_b)
        fire(idxs_b, rows_b, sem_b)
        pltpu.make_async_copy(tlo_hbm.at[idxs_a], rows_a, sem_a).wait()
        pltpu.sync_copy(rows_a, acc.at[idxd_a], add=True)

        @pl.when(p < np2 - 1)
        def _():
            b2 = e0 + (2 * p + 2) * _EB
            pltpu.sync_copy(src_hbm.at[pl.ds(b2, _EB)], idxs_a)
            pltpu.sync_copy(dst_hbm.at[pl.ds(b2, _EB)], idxd_a)
            fire(idxs_a, rows_a, sem_a)

        pltpu.make_async_copy(tlo_hbm.at[idxs_b], rows_b, sem_b).wait()
        pltpu.sync_copy(rows_b, acc.at[idxd_b], add=True)
        return carry

    lax.fori_loop(0, np2, body, 0)
    plsc.subcore_barrier()
    pltpu.sync_copy(acc.at[pl.ds(r0, _RPT)], out_hbm.at[cid, pl.ds(r0, _RPT)])


@functools.partial(
    pl.kernel, mesh=_mesh, compiler_params=_sc_params,
    out_type=[jax.ShapeDtypeStruct((_NCORE, _NP, 16), jnp.float32),
              jax.ShapeDtypeStruct((_NCORE, _NP, 16), jnp.float32)],
    scratch_types=[
        pltpu.VMEM((_EB,), jnp.int32),
        pltpu.VMEM((_EB,), jnp.int32),
        pltpu.VMEM((_EB,), jnp.int32),
        pltpu.VMEM((_EB,), jnp.int32),
        pltpu.VMEM((_EB, 16), jnp.float32),
        pltpu.VMEM((_EB, 16), jnp.float32),
        pltpu.VMEM((_EB, 16), jnp.float32),
        pltpu.VMEM((_EB, 16), jnp.float32),
        pltpu.VMEM((_EB, 16), jnp.float32),
        pltpu.VMEM((_EB, 16), jnp.float32),
        pltpu.VMEM((_EB, 16), jnp.float32),
        pltpu.VMEM((_EB, 16), jnp.float32),
        pltpu.VMEM((16,), jnp.float32),
        pltpu.VMEM((16,), jnp.float32),
        pltpu.VMEM_SHARED((_NP, 16), jnp.float32),
        pltpu.VMEM_SHARED((_NP, 16), jnp.float32),
        pltpu.SemaphoreType.DMA,
        pltpu.SemaphoreType.DMA,
    ],
)
def _sc_attn(src_hbm, dst_hbm, q_hbm, k_hbm, v_hbm, mq_hbm, mk_hbm, z_hbm,
             oe_hbm, oev_hbm,
             idxs_a, idxd_a, idxs_b, idxd_b, qr_a, kr_a, vr_a,
             qr_b, kr_b, vr_b, eb, evb, mq, mk, acc_e, acc_ev,
             sem_a, sem_b):
    sid, cid = _wid(), _cid()
    r0 = sid * _RPT
    pltpu.sync_copy(mq_hbm, mq)
    pltpu.sync_copy(mk_hbm, mk)
    pltpu.sync_copy(z_hbm.at[pl.ds(r0, _RPT)], acc_e.at[pl.ds(r0, _RPT)])
    pltpu.sync_copy(z_hbm.at[pl.ds(r0, _RPT)], acc_ev.at[pl.ds(r0, _RPT)])
    plsc.subcore_barrier()
    bvec = mq[...] * mk[...]
    e0 = cid * _EC + sid * (_TB_SPLIT * _EB)
    np2 = _TB_SPLIT // 2

    def fire(idxs, idxd, qr, kr, vr, sem):
        pltpu.async_copy(q_hbm.at[idxd], qr, sem)
        pltpu.async_copy(k_hbm.at[idxs], kr, sem)
        pltpu.async_copy(v_hbm.at[idxs], vr, sem)

    def drain(qr, kr, vr, sem):
        pltpu.make_async_copy(q_hbm.at[idxs_a], qr, sem).wait()
        pltpu.make_async_copy(q_hbm.at[idxs_a], kr, sem).wait()
        pltpu.make_async_copy(q_hbm.at[idxs_a], vr, sem).wait()

    def load_idx(base, idxs, idxd):
        pltpu.sync_copy(src_hbm.at[pl.ds(base, _EB)], idxs)
        pltpu.sync_copy(dst_hbm.at[pl.ds(base, _EB)], idxd)

    def compute_scatter(qr, kr, vr, idxd):
        def inner(jj, c):
            for u in range(4):
                j = 4 * jj + u
                e = jnp.exp(qr[j] * kr[j] - bvec)
                eb[j] = e
                evb[j] = e * vr[j]
            return c

        lax.fori_loop(0, _EB // 4, inner, 0)
        pltpu.sync_copy(eb, acc_e.at[idxd], add=True)
        pltpu.sync_copy(evb, acc_ev.at[idxd], add=True)

    load_idx(e0, idxs_a, idxd_a)
    fire(idxs_a, idxd_a, qr_a, kr_a, vr_a, sem_a)

    def body(p, carry):
        load_idx(e0 + (2 * p + 1) * _EB, idxs_b, idxd_b)
        fire(idxs_b, idxd_b, qr_b, kr_b, vr_b, sem_b)
        drain(qr_a, kr_a, vr_a, sem_a)
        compute_scatter(qr_a, kr_a, vr_a, idxd_a)

        @pl.when(p < np2 - 1)
        def _():
            load_idx(e0 + (2 * p + 2) * _EB, idxs_a, idxd_a)
            fire(idxs_a, idxd_a, qr_a, kr_a, vr_a, sem_a)

        drain(qr_b, kr_b, vr_b, sem_b)
        compute_scatter(qr_b, kr_b, vr_b, idxd_b)
        return carry

    lax.fori_loop(0, np2, body, 0)
    plsc.subcore_barrier()
    pltpu.sync_copy(acc_e.at[pl.ds(r0, _RPT)], oe_hbm.at[cid, pl.ds(r0, _RPT)])
    pltpu.sync_copy(acc_ev.at[pl.ds(r0, _RPT)],
                    oev_hbm.at[cid, pl.ds(r0, _RPT)])


def _ln(x, g, b):
    m = jnp.mean(x, axis=-1, keepdims=True)
    v = jnp.mean((x - m) * (x - m), axis=-1, keepdims=True)
    return (x - m) * lax.rsqrt(v + 1e-5) * g + b


def _t0_body(deg_ref, x_ref, dinv_ref, xs_ref):
    d = deg_ref[0, :, :1] + deg_ref[1, :, :1] + 1.0
    di = lax.rsqrt(d)
    dinv_ref[...] = di
    xs_ref[...] = jnp.concatenate(
        [di * x_ref[...], jnp.zeros((_RB, 8), jnp.float32)], axis=1)


def _t1_body(p1_ref, xs_ref, dinv_ref, w1_ref, b1_ref, lo_ref, hi_ref):
    di = dinv_ref[...]
    agg16 = di * (p1_ref[0] + p1_ref[1] + xs_ref[...])
    h1 = jax.nn.relu(agg16[:, :8] @ w1_ref[...] + b1_ref[...])
    h1s = di * h1
    lo_ref[...] = h1s[:, :32]
    hi_ref[...] = h1s[:, 32:]


def _t2_body(p2_ref, lo_ref, hi_ref, dinv_ref, w2_ref, b2_ref, w3_ref,
             b3_ref, wq_ref, bq_ref, wk_ref, bk_ref, wv_ref, bv_ref,
             h_ref, q_ref, k_ref, v_ref, mq_ref, mk_ref):
    di = dinv_ref[...]
    agg64 = di * (jnp.concatenate([p2_ref[0], p2_ref[1]], axis=1)
                  + jnp.concatenate([lo_ref[...], hi_ref[...]], axis=1))
    h2 = jax.nn.relu(agg64 @ w2_ref[...] + b2_ref[...])
    h = h2 @ w3_ref[...] + b3_ref[...]
    q = h @ wq_ref[...] + bq_ref[...]
    k = h @ wk_ref[...] + bk_ref[...]
    v = h @ wv_ref[...] + bv_ref[...]
    z = jnp.zeros((_RB, 11), jnp.float32)
    h_ref[...] = h
    q_ref[...] = jnp.concatenate([q, z], axis=1)
    k_ref[...] = jnp.concatenate([k, z], axis=1)
    v_ref[...] = jnp.concatenate([v, z], axis=1)
    z1 = jnp.zeros((1, 11), jnp.float32)
    qm = jnp.concatenate([jnp.max(jnp.abs(q), axis=0, keepdims=True), z1], 1)
    km = jnp.concatenate([jnp.max(jnp.abs(k), axis=0, keepdims=True), z1], 1)
    i = pl.program_id(0)

    @pl.when(i == 0)
    def _():
        mq_ref[...] = qm
        mk_ref[...] = km

    @pl.when(i > 0)
    def _():
        mq_ref[...] = jnp.maximum(mq_ref[...], qm)
        mk_ref[...] = jnp.maximum(mk_ref[...], km)


def _t3_body(oe_ref, oev_ref, h_ref, q_ref, k_ref, v_ref, mq_ref, mk_ref,
             wo_ref, bo_ref, g1_ref, be1_ref, wf1_ref, bf1_ref, wf2_ref,
             bf2_ref, g2_ref, be2_ref, out_ref):
    bvec = mq_ref[...] * mk_ref[...]
    es = jnp.exp(q_ref[...] * k_ref[...] - bvec)
    s5 = oe_ref[0, :, :5] + oe_ref[1, :, :5] + es[:, :5]
    ev5 = (oev_ref[0, :, :5] + oev_ref[1, :, :5]
           + es[:, :5] * v_ref[...][:, :5])
    ctx = ev5 / s5
    o = ctx @ wo_ref[...] + bo_ref[...]
    h1n = _ln(h_ref[...] + o, g1_ref[...], be1_ref[...])
    ff = jax.nn.relu(h1n @ wf1_ref[...] + bf1_ref[...]) @ wf2_ref[...] \
        + bf2_ref[...]
    out_ref[...] = _ln(h1n + ff, g2_ref[...], be2_ref[...])


def _row_bs(width):
    return pl.BlockSpec((_RB, width), lambda i: (i, 0))


def _pair_bs(width):
    return pl.BlockSpec((2, _RB, width), lambda i: (0, i, 0))


def _full_bs(shape):
    nd = len(shape)
    return pl.BlockSpec(shape, lambda i: (0,) * nd)


def _tc_call(body, in_specs, out_specs, out_shapes):
    return pl.pallas_call(
        body, grid=(_G,), in_specs=in_specs, out_specs=out_specs,
        out_shape=out_shapes)


def kernel(x, edge_index, batch, W1, b1, W2, b2, W3, b3, Wq, bq, Wk, bk,
           Wv, bv, Wo, bo, ln1_g, ln1_b, Wf1, bf1, Wf2, bf2, ln2_g, ln2_b):
    f32 = jnp.float32
    src = jnp.concatenate(
        [edge_index[0], jnp.zeros((_EPAD - _E,), jnp.int32)])
    dst = jnp.concatenate(
        [edge_index[1], jnp.full((_EPAD - _E,), _N, jnp.int32)])
    ones128 = jnp.ones((_EB, 16), f32)
    z16 = jnp.zeros((_NP, 16), f32)
    z32 = jnp.zeros((_NP, 32), f32)
    b1r = b1.reshape(1, -1)
    b2r = b2.reshape(1, -1)
    b3r = b3.reshape(1, -1)
    bqr, bkr, bvr, bor = (t.reshape(1, -1) for t in (bq, bk, bv, bo))
    bf1r, bf2r = bf1.reshape(1, -1), bf2.reshape(1, -1)
    g1r, be1r = ln1_g.reshape(1, -1), ln1_b.reshape(1, -1)
    g2r, be2r = ln2_g.reshape(1, -1), ln2_b.reshape(1, -1)

    deg2 = _sc_deg(dst, ones128, z16)

    dinv, xs16 = _tc_call(
        _t0_body,
        [_pair_bs(16), _row_bs(8)],
        [_row_bs(1), _row_bs(16)],
        [jax.ShapeDtypeStruct((_N, 1), f32),
         jax.ShapeDtypeStruct((_N, 16), f32)],
    )(deg2[:, :_N], x)

    p1 = _sc_agg16(src, dst, xs16, z16)

    h1lo, h1hi = _tc_call(
        _t1_body,
        [_pair_bs(16), _row_bs(16), _row_bs(1), _full_bs((8, 64)),
         _full_bs((1, 64))],
        [_row_bs(32), _row_bs(32)],
        [jax.ShapeDtypeStruct((_N, 32), f32),
         jax.ShapeDtypeStruct((_N, 32), f32)],
    )(p1[:, :_N], xs16, dinv, W1, b1r)

    p2 = _sc_agg64(src, dst, h1lo, h1hi, z32)

    h5, q16, k16, v16, mq, mk = _tc_call(
        _t2_body,
        [_pair_bs(32), _row_bs(32), _row_bs(32), _row_bs(1),
         _full_bs((64, 150)), _full_bs((1, 150)), _full_bs((150, 5)),
         _full_bs((1, 5)), _full_bs((5, 5)), _full_bs((1, 5)),
         _full_bs((5, 5)), _full_bs((1, 5)), _full_bs((5, 5)),
         _full_bs((1, 5))],
        [_row_bs(5), _row_bs(16), _row_bs(16), _row_bs(16),
         _full_bs((1, 16)), _full_bs((1, 16))],
        [jax.ShapeDtypeStruct((_N, 5), f32),
         jax.ShapeDtypeStruct((_N, 16), f32),
         jax.ShapeDtypeStruct((_N, 16), f32),
         jax.ShapeDtypeStruct((_N, 16), f32),
         jax.ShapeDtypeStruct((1, 16), f32),
         jax.ShapeDtypeStruct((1, 16), f32)],
    )(p2[:, :_N], h1lo, h1hi, dinv, W2, b2r, W3, b3r, Wq, bqr, Wk, bkr,
      Wv, bvr)

    oe, oev = _sc_attn(src, dst, q16, k16, v16, mq.reshape(16),
                       mk.reshape(16), z16)

    out = _tc_call(
        _t3_body,
        [_pair_bs(16), _pair_bs(16), _row_bs(5), _row_bs(16), _row_bs(16),
         _row_bs(16), _full_bs((1, 16)), _full_bs((1, 16)),
         _full_bs((5, 5)), _full_bs((1, 5)), _full_bs((1, 5)),
         _full_bs((1, 5)), _full_bs((5, 64)), _full_bs((1, 64)),
         _full_bs((64, 5)), _full_bs((1, 5)), _full_bs((1, 5)),
         _full_bs((1, 5))],
        [_row_bs(5)],
        [jax.ShapeDtypeStruct((_N, 5), f32)],
    )(oe[:, :_N], oev[:, :_N], h5, q16, k16, v16, mq, mk, Wo, bor, g1r,
      be1r, Wf1, bf1r, Wf2, bf2r, g2r, be2r)
    return out[0]
